# baseline (speedup 1.0000x reference)
import jax
import jax.numpy as jnp
from jax.experimental import pallas as pl
from jax.experimental.pallas import tpu as pltpu

_EPS = 1e-5


def _patches(x, th, tw, oh, ow):
    pieces = [x[:, a:a + oh, b:b + ow, :] for a in range(th) for b in range(tw)]
    return jnp.concatenate(pieces, axis=-1)


def _mm_affine(p, w, aff):
    bt, oh, ow, k = p.shape
    acc = jnp.dot(p.reshape(bt * oh * ow, k), w,
                  preferred_element_type=jnp.float32)
    acc = acc + aff[0:1, :]
    acc = jnp.maximum(acc, 0.0)
    acc = acc * aff[1:2, :] + aff[2:3, :]
    return acc.reshape(bt, oh, ow, w.shape[1])


def _s2d2(y):
    bt, h2, w2, c = y.shape
    h, w = h2 // 2, w2 // 2
    y = y.reshape(bt, h, 2, w2, c)
    pieces = []
    for r in range(2):
        t = y[:, :, r].reshape(bt, h, w, 2, c)
        for q in range(2):
            pieces.append(t[:, :, :, q])
    return jnp.concatenate(pieces, axis=-1)


def _d2s2(y):
    bt, h, w, c4 = y.shape
    c = c4 // 4
    ps = [y[..., i * c:(i + 1) * c] for i in range(4)]
    r0 = jnp.stack([ps[0], ps[1]], axis=3).reshape(bt, h, 2 * w, c)
    r1 = jnp.stack([ps[2], ps[3]], axis=3).reshape(bt, h, 2 * w, c)
    return jnp.stack([r0, r1], axis=2).reshape(bt, 2 * h, 2 * w, c)


def _net_kernel(xs_ref, w1_ref, a1_ref, w2_ref, a2_ref, w3_ref, a3_ref,
                w4_ref, a4_ref, w5_ref, a5_ref, w6_ref, b6_ref, o_ref):
    bt = xs_ref.shape[0]
    xs = xs_ref[...]

    y = _mm_affine(_patches(xs, 2, 2, 20, 20), w1_ref[...], a1_ref[...])
    y = _s2d2(y)
    y = _mm_affine(_patches(y, 2, 2, 9, 9), w2_ref[...], a2_ref[...])
    y = _mm_affine(_patches(y, 3, 3, 7, 7), w3_ref[...], a3_ref[...])
    y = jnp.pad(y, ((0, 0), (2, 2), (2, 2), (0, 0)))
    y = _mm_affine(_patches(y, 3, 3, 9, 9), w4_ref[...], a4_ref[...])
    y = jnp.pad(y, ((0, 0), (1, 1), (1, 1), (0, 0)))
    y = _mm_affine(_patches(y, 2, 2, 10, 10), w5_ref[...], a5_ref[...])
    y = _d2s2(y)
    y = jnp.pad(y, ((0, 0), (1, 1), (1, 1), (0, 0)))
    p = _patches(y, 2, 2, 21, 21)
    z = jnp.dot(p.reshape(bt * 21 * 21, 128), w6_ref[...],
                preferred_element_type=jnp.float32)
    z = z + b6_ref[0:1, :]
    z = z.reshape(bt, 21, 21, 16)
    m = jnp.max(z, axis=(1, 2, 3), keepdims=True)
    e = jnp.exp(z - m)
    s = jnp.sum(e, axis=(1, 2, 3), keepdims=True)
    o_ref[...] = (z - m - jnp.log(s)).astype(o_ref.dtype)


def _affine(bias, g, b, m, v, n_phases=1):
    scale = g / jnp.sqrt(v + _EPS)
    shift = b - m * scale
    rows = jnp.stack([jnp.tile(bias, n_phases), jnp.tile(scale, n_phases),
                      jnp.tile(shift, n_phases)], axis=0)
    return jnp.pad(rows, ((0, 5), (0, 0)))


def _conv_w_s2d(w, s):
    cout, cin, k, _ = w.shape
    wt = jnp.transpose(w, (2, 3, 1, 0))
    wt = wt.reshape(2, s, 2, s, cin, cout)
    wt = jnp.transpose(wt, (0, 2, 1, 3, 4, 5))
    return wt.reshape(4 * s * s * cin, cout)


def _conv_w_flat(w):
    cout, cin, kh, kw = w.shape
    return jnp.transpose(w, (2, 3, 1, 0)).reshape(kh * kw * cin, cout)


def _deconv_w_phase(w, s):
    cin, cout, k, _ = w.shape
    t = k // s
    w6 = w.reshape(cin, cout, t, s, t, s)
    w6 = jnp.flip(w6, axis=(2, 4))
    w6 = jnp.transpose(w6, (2, 4, 0, 3, 5, 1))
    return w6.reshape(t * t * cin, s * s * cout)


def kernel(x, conv1_w, conv1_b, conv2_w, conv2_b, conv3_w, conv3_b,
           deconv1_w, deconv1_b, deconv2_w, deconv2_b, deconv3_w, deconv3_b,
           bn1_g, bn1_b, bn1_m, bn1_v, bn2_g, bn2_b, bn2_m, bn2_v,
           bn3_g, bn3_b, bn3_m, bn3_v, bn4_g, bn4_b, bn4_m, bn4_v,
           bn5_g, bn5_b, bn5_m, bn5_v):
    B = x.shape[0]
    bt = next(t for t in (16, 8, 4, 2, 1) if B % t == 0)

    xs = jnp.transpose(x, (0, 2, 3, 1)).astype(jnp.float32)
    xs = xs.reshape(B, 21, 4, 21, 4, 4)
    xs = jnp.transpose(xs, (0, 1, 3, 2, 4, 5)).reshape(B, 21, 21, 64)

    w1 = _conv_w_s2d(conv1_w, 4)
    w2 = _conv_w_s2d(conv2_w, 2)
    w3 = _conv_w_flat(conv3_w)
    w4 = _deconv_w_phase(deconv1_w, 1)
    w5 = _deconv_w_phase(deconv2_w, 2)
    w6 = _deconv_w_phase(deconv3_w, 4)
    a1 = _affine(conv1_b, bn1_g, bn1_b, bn1_m, bn1_v)
    a2 = _affine(conv2_b, bn2_g, bn2_b, bn2_m, bn2_v)
    a3 = _affine(conv3_b, bn3_g, bn3_b, bn3_m, bn3_v)
    a4 = _affine(deconv1_b, bn4_g, bn4_b, bn4_m, bn4_v)
    a5 = _affine(deconv2_b, bn5_g, bn5_b, bn5_m, bn5_v, n_phases=4)
    b6 = jnp.pad(jnp.tile(deconv3_b, 16)[None, :], ((0, 7), (0, 0)))

    def w_spec(arr):
        return pl.BlockSpec(arr.shape, lambda i: (0,) * arr.ndim)

    z = pl.pallas_call(
        _net_kernel,
        out_shape=jax.ShapeDtypeStruct((B, 21, 21, 16), jnp.float32),
        grid=(B // bt,),
        in_specs=[
            pl.BlockSpec((bt, 21, 21, 64), lambda i: (i, 0, 0, 0)),
            w_spec(w1), w_spec(a1), w_spec(w2), w_spec(a2),
            w_spec(w3), w_spec(a3), w_spec(w4), w_spec(a4),
            w_spec(w5), w_spec(a5), w_spec(w6), w_spec(b6),
        ],
        out_specs=pl.BlockSpec((bt, 21, 21, 16), lambda i: (i, 0, 0, 0)),
        compiler_params=pltpu.CompilerParams(
            dimension_semantics=("parallel",)),
    )(xs, w1, a1, w2, a2, w3, a3, w4, a4, w5, a5, w6, b6)

    z = z.reshape(B, 21, 21, 4, 4)
    z = jnp.transpose(z, (0, 1, 3, 2, 4))
    return z.reshape(B, 84, 84)


# speedup vs baseline: 1.5811x; 1.5811x over previous
import jax
import jax.numpy as jnp
from jax.experimental import pallas as pl
from jax.experimental.pallas import tpu as pltpu

_EPS = 1e-5


def _patches(x, th, tw, oh, ow):
    pieces = [x[:, a:a + oh, b:b + ow, :] for a in range(th) for b in range(tw)]
    return jnp.concatenate(pieces, axis=-1)


def _mm(p, w):
    bt, oh, ow, k = p.shape
    return jnp.dot(p.reshape(bt * oh * ow, k), w,
                   preferred_element_type=jnp.float32)


def _affine(acc, aff):
    acc = acc + aff[0:1, :]
    acc = jnp.maximum(acc, 0.0)
    acc = acc * aff[1:2, :] + aff[2:3, :]
    return acc.astype(jnp.bfloat16)


def _s2d2(y):
    bt, h2, w2, c = y.shape
    h, w = h2 // 2, w2 // 2
    y = y.reshape(bt, h, 2, w2, c)
    pieces = []
    for r in range(2):
        t = y[:, :, r].reshape(bt, h, w, 2, c)
        for q in range(2):
            pieces.append(t[:, :, :, q])
    return jnp.concatenate(pieces, axis=-1)


def _d2s2(y):
    bt, h, w, c4 = y.shape
    c = c4 // 4
    ps = [y[..., i * c:(i + 1) * c] for i in range(4)]
    r0 = jnp.stack([ps[0], ps[1]], axis=3).reshape(bt, h, 2 * w, c)
    r1 = jnp.stack([ps[2], ps[3]], axis=3).reshape(bt, h, 2 * w, c)
    return jnp.stack([r0, r1], axis=2).reshape(bt, 2 * h, 2 * w, c)


def _net_kernel(xs_ref, w1_ref, a1_ref, w2_ref, a2_ref, w3_ref, a3_ref,
                w4_ref, a4_ref, w5_ref, a5_ref, w6_ref, b6_ref, o_ref):
    bt = xs_ref.shape[0]
    xs = xs_ref[...]

    y = _mm(_patches(xs, 2, 2, 20, 20), w1_ref[...])
    y = y.astype(jnp.bfloat16).reshape(bt, 20, 20, 32)
    y = _s2d2(y)
    y = _affine(y.astype(jnp.float32), a1_ref[...])
    y = _affine(_mm(_patches(y, 2, 2, 9, 9), w2_ref[...]), a2_ref[...])
    y = y.reshape(bt, 9, 9, 64)
    y = _affine(_mm(_patches(y, 3, 3, 7, 7), w3_ref[...]), a3_ref[...])
    y = y.reshape(bt, 7, 7, 64)
    y = jnp.pad(y, ((0, 0), (2, 2), (2, 2), (0, 0)))
    y = _affine(_mm(_patches(y, 3, 3, 9, 9), w4_ref[...]), a4_ref[...])
    y = y.reshape(bt, 9, 9, 64)
    y = jnp.pad(y, ((0, 0), (1, 1), (1, 1), (0, 0)))
    y = _affine(_mm(_patches(y, 2, 2, 10, 10), w5_ref[...]), a5_ref[...])
    y = _d2s2(y.reshape(bt, 10, 10, 128))
    y = jnp.pad(y, ((0, 0), (1, 1), (1, 1), (0, 0)))
    z = _mm(_patches(y, 2, 2, 21, 21), w6_ref[...])
    z = z.reshape(bt, 21, 21, 16)
    z = jnp.concatenate([z[:, :, j, :] for j in range(21)], axis=-1)
    z = z + b6_ref[0:1, :]
    m = jnp.max(z, axis=(1, 2), keepdims=True)
    e = jnp.exp(z - m)
    s = jnp.sum(e, axis=(1, 2), keepdims=True)
    o_ref[...] = (z - m - jnp.log(s)).astype(o_ref.dtype)


def _pack_affine(bias, g, b, m, v, n_phases=1):
    scale = g / jnp.sqrt(v + _EPS)
    shift = b - m * scale
    rows = jnp.stack([jnp.tile(bias, n_phases), jnp.tile(scale, n_phases),
                      jnp.tile(shift, n_phases)], axis=0)
    return jnp.pad(rows, ((0, 5), (0, 0)))


def _conv_w_s2d(w, s):
    cout, cin, k, _ = w.shape
    wt = jnp.transpose(w, (2, 3, 1, 0))
    wt = wt.reshape(2, s, 2, s, cin, cout)
    wt = jnp.transpose(wt, (0, 2, 1, 3, 4, 5))
    return wt.reshape(4 * s * s * cin, cout)


def _conv_w_flat(w):
    cout, cin, kh, kw = w.shape
    return jnp.transpose(w, (2, 3, 1, 0)).reshape(kh * kw * cin, cout)


def _deconv_w_phase(w, s):
    cin, cout, k, _ = w.shape
    t = k // s
    w6 = w.reshape(cin, cout, t, s, t, s)
    w6 = jnp.flip(w6, axis=(2, 4))
    w6 = jnp.transpose(w6, (2, 4, 0, 3, 5, 1))
    return w6.reshape(t * t * cin, s * s * cout)


def kernel(x, conv1_w, conv1_b, conv2_w, conv2_b, conv3_w, conv3_b,
           deconv1_w, deconv1_b, deconv2_w, deconv2_b, deconv3_w, deconv3_b,
           bn1_g, bn1_b, bn1_m, bn1_v, bn2_g, bn2_b, bn2_m, bn2_v,
           bn3_g, bn3_b, bn3_m, bn3_v, bn4_g, bn4_b, bn4_m, bn4_v,
           bn5_g, bn5_b, bn5_m, bn5_v):
    B = x.shape[0]
    bt = next(t for t in (8, 4, 2, 1) if B % t == 0)
    bf = jnp.bfloat16

    xs = jnp.transpose(x, (0, 2, 3, 1)).astype(jnp.float32)
    xs = xs.reshape(B, 21, 4, 21, 4, 4)
    xs = jnp.transpose(xs, (0, 1, 3, 2, 4, 5)).reshape(B, 21, 21, 64)
    xs = xs.astype(bf)

    w1 = _conv_w_s2d(conv1_w, 4).astype(bf)
    w2 = _conv_w_s2d(conv2_w, 2).astype(bf)
    w3 = _conv_w_flat(conv3_w).astype(bf)
    w4 = _deconv_w_phase(deconv1_w, 1).astype(bf)
    w5 = _deconv_w_phase(deconv2_w, 2).astype(bf)
    w6 = _deconv_w_phase(deconv3_w, 4).astype(bf)
    a1 = _pack_affine(conv1_b, bn1_g, bn1_b, bn1_m, bn1_v, n_phases=4)
    a2 = _pack_affine(conv2_b, bn2_g, bn2_b, bn2_m, bn2_v)
    a3 = _pack_affine(conv3_b, bn3_g, bn3_b, bn3_m, bn3_v)
    a4 = _pack_affine(deconv1_b, bn4_g, bn4_b, bn4_m, bn4_v)
    a5 = _pack_affine(deconv2_b, bn5_g, bn5_b, bn5_m, bn5_v, n_phases=4)
    b6 = jnp.pad(jnp.tile(deconv3_b, 21 * 16)[None, :], ((0, 7), (0, 0)))

    def w_spec(arr):
        return pl.BlockSpec(arr.shape, lambda i: (0,) * arr.ndim)

    z = pl.pallas_call(
        _net_kernel,
        out_shape=jax.ShapeDtypeStruct((B, 21, 21 * 16), jnp.float32),
        grid=(B // bt,),
        in_specs=[
            pl.BlockSpec((bt, 21, 21, 64), lambda i: (i, 0, 0, 0)),
            w_spec(w1), w_spec(a1), w_spec(w2), w_spec(a2),
            w_spec(w3), w_spec(a3), w_spec(w4), w_spec(a4),
            w_spec(w5), w_spec(a5), w_spec(w6), w_spec(b6),
        ],
        out_specs=pl.BlockSpec((bt, 21, 21 * 16), lambda i: (i, 0, 0)),
        compiler_params=pltpu.CompilerParams(
            dimension_semantics=("parallel",)),
    )(xs, w1, a1, w2, a2, w3, a3, w4, a4, w5, a5, w6, b6)

    z = z.reshape(B, 21, 21, 4, 4)
    z = jnp.transpose(z, (0, 1, 3, 2, 4))
    return z.reshape(B, 84, 84)


# speedup vs baseline: 2.1471x; 1.3580x over previous
import jax
import jax.numpy as jnp
from jax.experimental import pallas as pl
from jax.experimental.pallas import tpu as pltpu

_EPS = 1e-5


def _conv_taps(x, th, tw, oh, ow, w_ref, cin):
    bt = x.shape[2]
    acc = None
    for a in range(th):
        for b in range(tw):
            t = a * tw + b
            p = x[a:a + oh, b:b + ow].reshape(oh * ow * bt, cin)
            d = jnp.dot(p, w_ref[t * cin:(t + 1) * cin, :],
                        preferred_element_type=jnp.float32)
            acc = d if acc is None else acc + d
    return acc


def _affine2(acc, aff):
    acc = acc + aff[0:1, :]
    acc = jnp.maximum(acc, 0.0)
    acc = acc * aff[1:2, :] + aff[2:3, :]
    return acc.astype(jnp.bfloat16)


def _net_kernel(xs_ref, w1_ref, a1_ref, w2_ref, a2_ref, w3_ref, a3_ref,
                w4_ref, a4_ref, w5_ref, a5_ref, w6_ref, b6_ref, o_ref):
    bt = xs_ref.shape[2]
    xs = xs_ref[...]

    acc = _conv_taps(xs, 2, 2, 20, 20, w1_ref, 64)
    y = acc.astype(jnp.bfloat16).reshape(20, 20, bt, 32)
    pieces = []
    for r in range(2):
        t = y.reshape(10, 2, 20, bt, 32)[:, r]
        for q in range(2):
            pieces.append(t.reshape(10, 10, 2, bt, 32)[:, :, q])
    y = jnp.concatenate(pieces, axis=-1)
    y = _affine2(y.astype(jnp.float32).reshape(100 * bt, 128), a1_ref[...])
    y = y.reshape(10, 10, bt, 128)
    acc = _conv_taps(y, 2, 2, 9, 9, w2_ref, 128)
    y = _affine2(acc, a2_ref[...]).reshape(9, 9, bt, 64)
    acc = _conv_taps(y, 3, 3, 7, 7, w3_ref, 64)
    y = _affine2(acc, a3_ref[...]).reshape(7, 7, bt, 64)
    y = jnp.pad(y, ((2, 2), (2, 2), (0, 0), (0, 0)))
    acc = _conv_taps(y, 3, 3, 9, 9, w4_ref, 64)
    y = _affine2(acc, a4_ref[...]).reshape(9, 9, bt, 64)
    y = jnp.pad(y, ((1, 1), (1, 1), (0, 0), (0, 0)))
    acc = _conv_taps(y, 2, 2, 10, 10, w5_ref, 64)
    y = _affine2(acc, a5_ref[...]).reshape(10, 10, bt, 128)
    ps = [y[..., i * 32:(i + 1) * 32] for i in range(4)]
    r0 = jnp.stack([ps[0], ps[1]], axis=2).reshape(10, 20, bt, 32)
    r1 = jnp.stack([ps[2], ps[3]], axis=2).reshape(10, 20, bt, 32)
    y = jnp.stack([r0, r1], axis=1).reshape(20, 20, bt, 32)
    y = jnp.pad(y, ((1, 1), (1, 1), (0, 0), (0, 0)))
    z = _conv_taps(y, 2, 2, 21, 21, w6_ref, 32)
    z = z.reshape(21, 21, bt, 16)
    z = jnp.concatenate([z[:, j] for j in range(21)], axis=-1)
    z = z + b6_ref[...][0]
    m = jnp.max(z, axis=(0, 2), keepdims=True)
    e = jnp.exp(z - m)
    s = jnp.sum(e, axis=(0, 2), keepdims=True)
    o_ref[...] = (z - m - jnp.log(s)).astype(o_ref.dtype)


def _pack_affine(bias, g, b, m, v, n_phases=1):
    scale = g / jnp.sqrt(v + _EPS)
    shift = b - m * scale
    rows = jnp.stack([jnp.tile(bias, n_phases), jnp.tile(scale, n_phases),
                      jnp.tile(shift, n_phases)], axis=0)
    return jnp.pad(rows, ((0, 5), (0, 0)))


def _conv_w_s2d(w, s):
    cout, cin, k, _ = w.shape
    wt = jnp.transpose(w, (2, 3, 1, 0))
    wt = wt.reshape(2, s, 2, s, cin, cout)
    wt = jnp.transpose(wt, (0, 2, 1, 3, 4, 5))
    return wt.reshape(4 * s * s * cin, cout)


def _conv_w_flat(w):
    cout, cin, kh, kw = w.shape
    return jnp.transpose(w, (2, 3, 1, 0)).reshape(kh * kw * cin, cout)


def _deconv_w_phase(w, s):
    cin, cout, k, _ = w.shape
    t = k // s
    w6 = w.reshape(cin, cout, t, s, t, s)
    w6 = jnp.flip(w6, axis=(2, 4))
    w6 = jnp.transpose(w6, (2, 4, 0, 3, 5, 1))
    return w6.reshape(t * t * cin, s * s * cout)


def kernel(x, conv1_w, conv1_b, conv2_w, conv2_b, conv3_w, conv3_b,
           deconv1_w, deconv1_b, deconv2_w, deconv2_b, deconv3_w, deconv3_b,
           bn1_g, bn1_b, bn1_m, bn1_v, bn2_g, bn2_b, bn2_m, bn2_v,
           bn3_g, bn3_b, bn3_m, bn3_v, bn4_g, bn4_b, bn4_m, bn4_v,
           bn5_g, bn5_b, bn5_m, bn5_v):
    B = x.shape[0]
    bt = 16 if B % 16 == 0 else B
    bf = jnp.bfloat16

    xs = jnp.transpose(x, (0, 2, 3, 1)).astype(jnp.float32)
    xs = xs.reshape(B, 21, 4, 21, 4, 4)
    xs = jnp.transpose(xs, (1, 3, 0, 2, 4, 5)).reshape(21, 21, B, 64)
    xs = xs.astype(bf)

    w1 = _conv_w_s2d(conv1_w, 4).astype(bf)
    w2 = _conv_w_s2d(conv2_w, 2).astype(bf)
    w3 = _conv_w_flat(conv3_w).astype(bf)
    w4 = _deconv_w_phase(deconv1_w, 1).astype(bf)
    w5 = _deconv_w_phase(deconv2_w, 2).astype(bf)
    w6 = _deconv_w_phase(deconv3_w, 4).astype(bf)
    a1 = _pack_affine(conv1_b, bn1_g, bn1_b, bn1_m, bn1_v, n_phases=4)
    a2 = _pack_affine(conv2_b, bn2_g, bn2_b, bn2_m, bn2_v)
    a3 = _pack_affine(conv3_b, bn3_g, bn3_b, bn3_m, bn3_v)
    a4 = _pack_affine(deconv1_b, bn4_g, bn4_b, bn4_m, bn4_v)
    a5 = _pack_affine(deconv2_b, bn5_g, bn5_b, bn5_m, bn5_v, n_phases=4)
    b6 = jnp.pad(jnp.tile(deconv3_b, 21 * 16)[None, :], ((0, 7), (0, 0)))

    def w_spec(arr):
        return pl.BlockSpec(arr.shape, lambda i: (0,) * arr.ndim)

    z = pl.pallas_call(
        _net_kernel,
        out_shape=jax.ShapeDtypeStruct((21, B, 21 * 16), jnp.float32),
        grid=(B // bt,),
        in_specs=[
            pl.BlockSpec((21, 21, bt, 64), lambda i: (0, 0, i, 0)),
            w_spec(w1), w_spec(a1), w_spec(w2), w_spec(a2),
            w_spec(w3), w_spec(a3), w_spec(w4), w_spec(a4),
            w_spec(w5), w_spec(a5), w_spec(w6), w_spec(b6),
        ],
        out_specs=pl.BlockSpec((21, bt, 21 * 16), lambda i: (0, i, 0)),
        compiler_params=pltpu.CompilerParams(
            dimension_semantics=("parallel",)),
    )(xs, w1, a1, w2, a2, w3, a3, w4, a4, w5, a5, w6, b6)

    z = z.reshape(21, B, 21, 4, 4)
    z = jnp.transpose(z, (1, 0, 3, 2, 4))
    return z.reshape(B, 84, 84)


# speedup vs baseline: 2.7195x; 1.2666x over previous
import jax
import jax.numpy as jnp
from jax.experimental import pallas as pl
from jax.experimental.pallas import tpu as pltpu

_EPS = 1e-5


def _conv_taps(x, th, tw, oh, ow, w_ref, cin):
    bt = x.shape[2]
    acc = None
    for a in range(th):
        for b in range(tw):
            t = a * tw + b
            p = x[a:a + oh, b:b + ow].reshape(oh * ow * bt, cin)
            d = jnp.dot(p, w_ref[t * cin:(t + 1) * cin, :],
                        preferred_element_type=jnp.float32)
            acc = d if acc is None else acc + d
    return acc


def _conv_cat(x, th, tw, oh, ow, w_ref, cin):
    bt = x.shape[2]
    pieces = [x[a:a + oh, b:b + ow] for a in range(th) for b in range(tw)]
    p = jnp.concatenate(pieces, axis=-1)
    return jnp.dot(p.reshape(oh * ow * bt, th * tw * cin), w_ref[...],
                   preferred_element_type=jnp.float32)


def _affine2(acc, aff):
    acc = acc + aff[0:1, :]
    acc = jnp.maximum(acc, 0.0)
    acc = acc * aff[1:2, :] + aff[2:3, :]
    return acc.astype(jnp.bfloat16)


def _net_kernel(xs_ref, w1_ref, a1_ref, w2_ref, a2_ref, w3_ref, a3_ref,
                w4_ref, a4_ref, w5_ref, a5_ref, w6_ref, b6_ref, o_ref):
    bt = xs_ref.shape[2]
    xs = xs_ref[...]

    acc = _conv_cat(xs, 2, 2, 20, 20, w1_ref, 64)
    y = acc.astype(jnp.bfloat16).reshape(20, 20, bt, 32)
    pieces = []
    for r in range(2):
        t = y.reshape(10, 2, 20, bt, 32)[:, r]
        for q in range(2):
            pieces.append(t.reshape(10, 10, 2, bt, 32)[:, :, q])
    y = jnp.concatenate(pieces, axis=-1)
    y = _affine2(y.astype(jnp.float32).reshape(100 * bt, 128), a1_ref[...])
    y = y.reshape(10, 10, bt, 128)
    acc = _conv_taps(y, 2, 2, 9, 9, w2_ref, 128)
    y = _affine2(acc, a2_ref[...]).reshape(9, 9, bt, 64)
    acc = _conv_taps(y, 3, 3, 7, 7, w3_ref, 64)
    y = _affine2(acc, a3_ref[...]).reshape(7, 7, bt, 64)
    y = jnp.pad(y, ((2, 2), (2, 2), (0, 0), (0, 0)))
    acc = _conv_taps(y, 3, 3, 9, 9, w4_ref, 64)
    y = _affine2(acc, a4_ref[...]).reshape(9, 9, bt, 64)
    y = jnp.pad(y, ((1, 1), (1, 1), (0, 0), (0, 0)))
    acc = _conv_taps(y, 2, 2, 10, 10, w5_ref, 64)
    y = _affine2(acc, a5_ref[...]).reshape(10, 10, bt, 128)
    ps = [y[..., i * 32:(i + 1) * 32] for i in range(4)]
    r0 = jnp.stack([ps[0], ps[1]], axis=2).reshape(10, 20, bt, 32)
    r1 = jnp.stack([ps[2], ps[3]], axis=2).reshape(10, 20, bt, 32)
    y = jnp.stack([r0, r1], axis=1).reshape(20, 20, bt, 32)
    y = jnp.pad(y, ((1, 1), (1, 1), (0, 0), (0, 0)))
    z = _conv_cat(y, 2, 2, 21, 21, w6_ref, 32)
    z = z.reshape(21, 21, bt, 16)
    z = jnp.concatenate([z[:, j] for j in range(21)], axis=-1)
    z = z + b6_ref[...][0]
    m = jnp.max(z, axis=(0, 2), keepdims=True)
    e = jnp.exp(z - m)
    s = jnp.sum(e, axis=(0, 2), keepdims=True)
    o_ref[...] = (z - m - jnp.log(s)).astype(o_ref.dtype)


def _pack_affine(bias, g, b, m, v, n_phases=1):
    scale = g / jnp.sqrt(v + _EPS)
    shift = b - m * scale
    rows = jnp.stack([jnp.tile(bias, n_phases), jnp.tile(scale, n_phases),
                      jnp.tile(shift, n_phases)], axis=0)
    return jnp.pad(rows, ((0, 5), (0, 0)))


def _conv_w_s2d(w, s):
    cout, cin, k, _ = w.shape
    wt = jnp.transpose(w, (2, 3, 1, 0))
    wt = wt.reshape(2, s, 2, s, cin, cout)
    wt = jnp.transpose(wt, (0, 2, 1, 3, 4, 5))
    return wt.reshape(4 * s * s * cin, cout)


def _conv_w_flat(w):
    cout, cin, kh, kw = w.shape
    return jnp.transpose(w, (2, 3, 1, 0)).reshape(kh * kw * cin, cout)


def _deconv_w_phase(w, s):
    cin, cout, k, _ = w.shape
    t = k // s
    w6 = w.reshape(cin, cout, t, s, t, s)
    w6 = jnp.flip(w6, axis=(2, 4))
    w6 = jnp.transpose(w6, (2, 4, 0, 3, 5, 1))
    return w6.reshape(t * t * cin, s * s * cout)


def kernel(x, conv1_w, conv1_b, conv2_w, conv2_b, conv3_w, conv3_b,
           deconv1_w, deconv1_b, deconv2_w, deconv2_b, deconv3_w, deconv3_b,
           bn1_g, bn1_b, bn1_m, bn1_v, bn2_g, bn2_b, bn2_m, bn2_v,
           bn3_g, bn3_b, bn3_m, bn3_v, bn4_g, bn4_b, bn4_m, bn4_v,
           bn5_g, bn5_b, bn5_m, bn5_v):
    B = x.shape[0]
    bt = 16 if B % 16 == 0 else B
    bf = jnp.bfloat16

    xs = jnp.transpose(x, (0, 2, 3, 1)).astype(jnp.float32)
    xs = xs.reshape(B, 21, 4, 21, 4, 4)
    xs = jnp.transpose(xs, (1, 3, 0, 2, 4, 5)).reshape(21, 21, B, 64)
    xs = xs.astype(bf)

    w1 = _conv_w_s2d(conv1_w, 4).astype(bf)
    w2 = _conv_w_s2d(conv2_w, 2).astype(bf)
    w3 = _conv_w_flat(conv3_w).astype(bf)
    w4 = _deconv_w_phase(deconv1_w, 1).astype(bf)
    w5 = _deconv_w_phase(deconv2_w, 2).astype(bf)
    w6 = _deconv_w_phase(deconv3_w, 4).astype(bf)
    a1 = _pack_affine(conv1_b, bn1_g, bn1_b, bn1_m, bn1_v, n_phases=4)
    a2 = _pack_affine(conv2_b, bn2_g, bn2_b, bn2_m, bn2_v)
    a3 = _pack_affine(conv3_b, bn3_g, bn3_b, bn3_m, bn3_v)
    a4 = _pack_affine(deconv1_b, bn4_g, bn4_b, bn4_m, bn4_v)
    a5 = _pack_affine(deconv2_b, bn5_g, bn5_b, bn5_m, bn5_v, n_phases=4)
    b6 = jnp.pad(jnp.tile(deconv3_b, 21 * 16)[None, :], ((0, 7), (0, 0)))

    def w_spec(arr):
        return pl.BlockSpec(arr.shape, lambda i: (0,) * arr.ndim)

    z = pl.pallas_call(
        _net_kernel,
        out_shape=jax.ShapeDtypeStruct((21, B, 21 * 16), jnp.float32),
        grid=(B // bt,),
        in_specs=[
            pl.BlockSpec((21, 21, bt, 64), lambda i: (0, 0, i, 0)),
            w_spec(w1), w_spec(a1), w_spec(w2), w_spec(a2),
            w_spec(w3), w_spec(a3), w_spec(w4), w_spec(a4),
            w_spec(w5), w_spec(a5), w_spec(w6), w_spec(b6),
        ],
        out_specs=pl.BlockSpec((21, bt, 21 * 16), lambda i: (0, i, 0)),
        compiler_params=pltpu.CompilerParams(
            dimension_semantics=("parallel",)),
    )(xs, w1, a1, w2, a2, w3, a3, w4, a4, w5, a5, w6, b6)

    z = z.reshape(21, B, 21, 4, 4)
    z = jnp.transpose(z, (1, 0, 3, 2, 4))
    return z.reshape(B, 84, 84)


# speedup vs baseline: 3.1672x; 1.1646x over previous
import jax
import jax.numpy as jnp
from jax.experimental import pallas as pl
from jax.experimental.pallas import tpu as pltpu

_EPS = 1e-5


def _conv_taps(x, th, tw, oh, ow, w_ref, cin):
    bt = x.shape[2]
    acc = None
    for a in range(th):
        for b in range(tw):
            t = a * tw + b
            p = x[a:a + oh, b:b + ow].reshape(oh * ow * bt, cin)
            d = jnp.dot(p, w_ref[t * cin:(t + 1) * cin, :],
                        preferred_element_type=jnp.float32)
            acc = d if acc is None else acc + d
    return acc


def _conv_cat(x, th, tw, oh, ow, w_ref, cin):
    bt = x.shape[2]
    pieces = [x[a:a + oh, b:b + ow] for a in range(th) for b in range(tw)]
    p = jnp.concatenate(pieces, axis=-1)
    return jnp.dot(p.reshape(oh * ow * bt, th * tw * cin), w_ref[...],
                   preferred_element_type=jnp.float32)


def _affine2(acc, aff):
    acc = acc + aff[0:1, :]
    acc = jnp.maximum(acc, 0.0)
    acc = acc * aff[1:2, :] + aff[2:3, :]
    return acc.astype(jnp.bfloat16)


def _net_kernel(xs_ref, w1_ref, a1_ref, w2_ref, a2_ref, w3_ref, a3_ref,
                w4_ref, a4_ref, w5_ref, a5_ref, w6_ref, b6_ref, o_ref):
    bt = xs_ref.shape[2]
    xs = xs_ref[...]

    acc = _conv_cat(xs, 2, 2, 20, 20, w1_ref, 64)
    y = acc.astype(jnp.bfloat16).reshape(20, 20, bt, 32)
    pieces = []
    for r in range(2):
        t = y.reshape(10, 2, 20, bt, 32)[:, r]
        for q in range(2):
            pieces.append(t.reshape(10, 10, 2, bt, 32)[:, :, q])
    y = jnp.concatenate(pieces, axis=-1)
    y = _affine2(y.astype(jnp.float32).reshape(100 * bt, 128), a1_ref[...])
    y = y.reshape(10, 10, bt, 128)
    acc = _conv_cat(y, 2, 2, 9, 9, w2_ref, 128)
    y = _affine2(acc, a2_ref[...]).reshape(9, 9, bt, 64)
    acc = _conv_cat(y, 3, 3, 7, 7, w3_ref, 64)
    y = _affine2(acc, a3_ref[...]).reshape(7, 7, bt, 64)
    y = jnp.pad(y, ((2, 2), (2, 2), (0, 0), (0, 0)))
    acc = _conv_cat(y, 3, 3, 9, 9, w4_ref, 64)
    y = _affine2(acc, a4_ref[...]).reshape(9, 9, bt, 64)
    y = jnp.pad(y, ((1, 1), (1, 1), (0, 0), (0, 0)))
    acc = _conv_cat(y, 2, 2, 10, 10, w5_ref, 64)
    y = _affine2(acc, a5_ref[...]).reshape(10, 10, bt, 128)
    ps = [y[..., i * 32:(i + 1) * 32] for i in range(4)]
    r0 = jnp.stack([ps[0], ps[1]], axis=2).reshape(10, 20, bt, 32)
    r1 = jnp.stack([ps[2], ps[3]], axis=2).reshape(10, 20, bt, 32)
    y = jnp.stack([r0, r1], axis=1).reshape(20, 20, bt, 32)
    y = jnp.pad(y, ((1, 1), (1, 1), (0, 0), (0, 0)))
    z = _conv_cat(y, 2, 2, 21, 21, w6_ref, 32)
    z = z.reshape(21, 21, bt, 16)
    z = jnp.concatenate([z[:, j] for j in range(21)], axis=-1)
    z = z + b6_ref[...][0]
    m = jnp.max(z, axis=(0, 2), keepdims=True)
    e = jnp.exp(z - m)
    s = jnp.sum(e, axis=(0, 2), keepdims=True)
    o_ref[...] = (z - m - jnp.log(s)).astype(o_ref.dtype)


def _pack_affine(bias, g, b, m, v, n_phases=1):
    scale = g / jnp.sqrt(v + _EPS)
    shift = b - m * scale
    rows = jnp.stack([jnp.tile(bias, n_phases), jnp.tile(scale, n_phases),
                      jnp.tile(shift, n_phases)], axis=0)
    return jnp.pad(rows, ((0, 5), (0, 0)))


def _conv_w_s2d(w, s):
    cout, cin, k, _ = w.shape
    wt = jnp.transpose(w, (2, 3, 1, 0))
    wt = wt.reshape(2, s, 2, s, cin, cout)
    wt = jnp.transpose(wt, (0, 2, 1, 3, 4, 5))
    return wt.reshape(4 * s * s * cin, cout)


def _conv_w_flat(w):
    cout, cin, kh, kw = w.shape
    return jnp.transpose(w, (2, 3, 1, 0)).reshape(kh * kw * cin, cout)


def _deconv_w_phase(w, s):
    cin, cout, k, _ = w.shape
    t = k // s
    w6 = w.reshape(cin, cout, t, s, t, s)
    w6 = jnp.flip(w6, axis=(2, 4))
    w6 = jnp.transpose(w6, (2, 4, 0, 3, 5, 1))
    return w6.reshape(t * t * cin, s * s * cout)


def kernel(x, conv1_w, conv1_b, conv2_w, conv2_b, conv3_w, conv3_b,
           deconv1_w, deconv1_b, deconv2_w, deconv2_b, deconv3_w, deconv3_b,
           bn1_g, bn1_b, bn1_m, bn1_v, bn2_g, bn2_b, bn2_m, bn2_v,
           bn3_g, bn3_b, bn3_m, bn3_v, bn4_g, bn4_b, bn4_m, bn4_v,
           bn5_g, bn5_b, bn5_m, bn5_v):
    B = x.shape[0]
    bt = 16 if B % 16 == 0 else B
    bf = jnp.bfloat16

    xs = jnp.transpose(x, (0, 2, 3, 1)).astype(jnp.float32)
    xs = xs.reshape(B, 21, 4, 21, 4, 4)
    xs = jnp.transpose(xs, (1, 3, 0, 2, 4, 5)).reshape(21, 21, B, 64)
    xs = xs.astype(bf)

    w1 = _conv_w_s2d(conv1_w, 4).astype(bf)
    w2 = _conv_w_s2d(conv2_w, 2).astype(bf)
    w3 = _conv_w_flat(conv3_w).astype(bf)
    w4 = _deconv_w_phase(deconv1_w, 1).astype(bf)
    w5 = _deconv_w_phase(deconv2_w, 2).astype(bf)
    w6 = _deconv_w_phase(deconv3_w, 4).astype(bf)
    a1 = _pack_affine(conv1_b, bn1_g, bn1_b, bn1_m, bn1_v, n_phases=4)
    a2 = _pack_affine(conv2_b, bn2_g, bn2_b, bn2_m, bn2_v)
    a3 = _pack_affine(conv3_b, bn3_g, bn3_b, bn3_m, bn3_v)
    a4 = _pack_affine(deconv1_b, bn4_g, bn4_b, bn4_m, bn4_v)
    a5 = _pack_affine(deconv2_b, bn5_g, bn5_b, bn5_m, bn5_v, n_phases=4)
    b6 = jnp.pad(jnp.tile(deconv3_b, 21 * 16)[None, :], ((0, 7), (0, 0)))

    def w_spec(arr):
        return pl.BlockSpec(arr.shape, lambda i: (0,) * arr.ndim)

    z = pl.pallas_call(
        _net_kernel,
        out_shape=jax.ShapeDtypeStruct((21, B, 21 * 16), jnp.float32),
        grid=(B // bt,),
        in_specs=[
            pl.BlockSpec((21, 21, bt, 64), lambda i: (0, 0, i, 0)),
            w_spec(w1), w_spec(a1), w_spec(w2), w_spec(a2),
            w_spec(w3), w_spec(a3), w_spec(w4), w_spec(a4),
            w_spec(w5), w_spec(a5), w_spec(w6), w_spec(b6),
        ],
        out_specs=pl.BlockSpec((21, bt, 21 * 16), lambda i: (0, i, 0)),
        compiler_params=pltpu.CompilerParams(
            dimension_semantics=("parallel",)),
    )(xs, w1, a1, w2, a2, w3, a3, w4, a4, w5, a5, w6, b6)

    z = z.reshape(21, B, 21, 4, 4)
    z = jnp.transpose(z, (1, 0, 3, 2, 4))
    return z.reshape(B, 84, 84)


# speedup vs baseline: 3.3174x; 1.0474x over previous
import jax
import jax.numpy as jnp
from jax.experimental import pallas as pl
from jax.experimental.pallas import tpu as pltpu

_EPS = 1e-5


def _conv_taps(x, th, tw, oh, ow, w_ref, cin):
    bt = x.shape[2]
    acc = None
    for a in range(th):
        for b in range(tw):
            t = a * tw + b
            p = x[a:a + oh, b:b + ow].reshape(oh * ow * bt, cin)
            d = jnp.dot(p, w_ref[t * cin:(t + 1) * cin, :],
                        preferred_element_type=jnp.float32)
            acc = d if acc is None else acc + d
    return acc


def _conv_cat(x, th, tw, oh, ow, w_ref, cin):
    bt = x.shape[2]
    pieces = [x[a:a + oh, b:b + ow] for a in range(th) for b in range(tw)]
    p = jnp.concatenate(pieces, axis=-1)
    return jnp.dot(p.reshape(oh * ow * bt, th * tw * cin), w_ref[...],
                   preferred_element_type=jnp.float32)


def _affine2(acc, aff):
    acc = acc + aff[0:1, :]
    acc = jnp.maximum(acc, 0.0)
    acc = acc * aff[1:2, :] + aff[2:3, :]
    return acc.astype(jnp.bfloat16)


def _net_kernel(xs_ref, w1_ref, a1_ref, w2_ref, a2_ref, w3_ref, a3_ref,
                w4_ref, a4_ref, w5_ref, a5_ref, w6_ref, b6_ref, o_ref):
    bt = xs_ref.shape[2]
    xs = xs_ref[...]

    acc = _conv_cat(xs, 2, 2, 20, 20, w1_ref, 64)
    y = acc.astype(jnp.bfloat16).reshape(20, 20, bt, 32)
    pieces = []
    for r in range(2):
        t = y.reshape(10, 2, 20, bt, 32)[:, r]
        for q in range(2):
            pieces.append(t.reshape(10, 10, 2, bt, 32)[:, :, q])
    y = jnp.concatenate(pieces, axis=-1)
    y = _affine2(y.astype(jnp.float32).reshape(100 * bt, 128), a1_ref[...])
    y = y.reshape(10, 10, bt, 128)
    acc = _conv_cat(y, 2, 2, 9, 9, w2_ref, 128)
    y = _affine2(acc, a2_ref[...]).reshape(9, 9, bt, 64)
    acc = _conv_cat(y, 3, 3, 7, 7, w3_ref, 64)
    y = _affine2(acc, a3_ref[...]).reshape(7, 7, bt, 64)
    y = jnp.pad(y, ((2, 2), (2, 2), (0, 0), (0, 0)))
    acc = _conv_cat(y, 3, 3, 9, 9, w4_ref, 64)
    y = _affine2(acc, a4_ref[...]).reshape(9, 9, bt, 64)
    y = jnp.pad(y, ((1, 1), (1, 1), (0, 0), (0, 0)))
    acc = _conv_cat(y, 2, 2, 10, 10, w5_ref, 64)
    y = _affine2(acc, a5_ref[...]).reshape(10, 10, bt, 128)
    ps = [y[..., i * 32:(i + 1) * 32] for i in range(4)]
    r0 = jnp.stack([ps[0], ps[1]], axis=2).reshape(10, 20, bt, 32)
    r1 = jnp.stack([ps[2], ps[3]], axis=2).reshape(10, 20, bt, 32)
    y = jnp.stack([r0, r1], axis=1).reshape(20, 20, bt, 32)
    y = jnp.pad(y, ((1, 1), (1, 1), (0, 0), (0, 0)))
    z = _conv_cat(y, 2, 2, 21, 21, w6_ref, 32)
    z = z.reshape(21, 21, bt, 16)
    z = jnp.concatenate([z[:, j] for j in range(21)], axis=-1)
    z = z + b6_ref[...][0]
    m = jnp.max(z, axis=(0, 2), keepdims=True)
    e = jnp.exp(z - m)
    s = jnp.sum(e, axis=(0, 2), keepdims=True)
    o_ref[...] = (z - m - jnp.log(s)).astype(o_ref.dtype)


def _pack_affine(bias, g, b, m, v, n_phases=1):
    scale = g / jnp.sqrt(v + _EPS)
    shift = b - m * scale
    rows = jnp.stack([jnp.tile(bias, n_phases), jnp.tile(scale, n_phases),
                      jnp.tile(shift, n_phases)], axis=0)
    return jnp.pad(rows, ((0, 5), (0, 0)))


def _conv_w_s2d(w, s):
    cout, cin, k, _ = w.shape
    wt = jnp.transpose(w, (2, 3, 1, 0))
    wt = wt.reshape(2, s, 2, s, cin, cout)
    wt = jnp.transpose(wt, (0, 2, 1, 3, 4, 5))
    return wt.reshape(4 * s * s * cin, cout)


def _conv_w_flat(w):
    cout, cin, kh, kw = w.shape
    return jnp.transpose(w, (2, 3, 1, 0)).reshape(kh * kw * cin, cout)


def _deconv_w_phase(w, s):
    cin, cout, k, _ = w.shape
    t = k // s
    w6 = w.reshape(cin, cout, t, s, t, s)
    w6 = jnp.flip(w6, axis=(2, 4))
    w6 = jnp.transpose(w6, (2, 4, 0, 3, 5, 1))
    return w6.reshape(t * t * cin, s * s * cout)


def kernel(x, conv1_w, conv1_b, conv2_w, conv2_b, conv3_w, conv3_b,
           deconv1_w, deconv1_b, deconv2_w, deconv2_b, deconv3_w, deconv3_b,
           bn1_g, bn1_b, bn1_m, bn1_v, bn2_g, bn2_b, bn2_m, bn2_v,
           bn3_g, bn3_b, bn3_m, bn3_v, bn4_g, bn4_b, bn4_m, bn4_v,
           bn5_g, bn5_b, bn5_m, bn5_v):
    B = x.shape[0]
    bt = 32 if B % 32 == 0 else (16 if B % 16 == 0 else B)
    bf = jnp.bfloat16

    xs = jnp.transpose(x, (0, 2, 3, 1)).astype(jnp.float32)
    xs = xs.reshape(B, 21, 4, 21, 4, 4)
    xs = jnp.transpose(xs, (1, 3, 0, 2, 4, 5)).reshape(21, 21, B, 64)
    xs = xs.astype(bf)

    w1 = _conv_w_s2d(conv1_w, 4).astype(bf)
    w2 = _conv_w_s2d(conv2_w, 2).astype(bf)
    w3 = _conv_w_flat(conv3_w).astype(bf)
    w4 = _deconv_w_phase(deconv1_w, 1).astype(bf)
    w5 = _deconv_w_phase(deconv2_w, 2).astype(bf)
    w6 = _deconv_w_phase(deconv3_w, 4).astype(bf)
    a1 = _pack_affine(conv1_b, bn1_g, bn1_b, bn1_m, bn1_v, n_phases=4)
    a2 = _pack_affine(conv2_b, bn2_g, bn2_b, bn2_m, bn2_v)
    a3 = _pack_affine(conv3_b, bn3_g, bn3_b, bn3_m, bn3_v)
    a4 = _pack_affine(deconv1_b, bn4_g, bn4_b, bn4_m, bn4_v)
    a5 = _pack_affine(deconv2_b, bn5_g, bn5_b, bn5_m, bn5_v, n_phases=4)
    b6 = jnp.pad(jnp.tile(deconv3_b, 21 * 16)[None, :], ((0, 7), (0, 0)))

    def w_spec(arr):
        return pl.BlockSpec(arr.shape, lambda i: (0,) * arr.ndim)

    z = pl.pallas_call(
        _net_kernel,
        out_shape=jax.ShapeDtypeStruct((21, B, 21 * 16), jnp.float32),
        grid=(B // bt,),
        in_specs=[
            pl.BlockSpec((21, 21, bt, 64), lambda i: (0, 0, i, 0)),
            w_spec(w1), w_spec(a1), w_spec(w2), w_spec(a2),
            w_spec(w3), w_spec(a3), w_spec(w4), w_spec(a4),
            w_spec(w5), w_spec(a5), w_spec(w6), w_spec(b6),
        ],
        out_specs=pl.BlockSpec((21, bt, 21 * 16), lambda i: (0, i, 0)),
        compiler_params=pltpu.CompilerParams(
            dimension_semantics=("parallel",)),
    )(xs, w1, a1, w2, a2, w3, a3, w4, a4, w5, a5, w6, b6)

    z = z.reshape(21, B, 21, 4, 4)
    z = jnp.transpose(z, (1, 0, 3, 2, 4))
    return z.reshape(B, 84, 84)


# speedup vs baseline: 3.3498x; 1.0098x over previous
import jax
import jax.numpy as jnp
from jax.experimental import pallas as pl
from jax.experimental.pallas import tpu as pltpu

_EPS = 1e-5


def _conv_taps(x, th, tw, oh, ow, w_ref, cin):
    bt = x.shape[2]
    acc = None
    for a in range(th):
        for b in range(tw):
            t = a * tw + b
            p = x[a:a + oh, b:b + ow].reshape(oh * ow * bt, cin)
            d = jnp.dot(p, w_ref[t * cin:(t + 1) * cin, :],
                        preferred_element_type=jnp.float32)
            acc = d if acc is None else acc + d
    return acc


def _conv_cat(x, th, tw, oh, ow, w_ref, cin):
    bt = x.shape[2]
    pieces = [x[a:a + oh, b:b + ow] for a in range(th) for b in range(tw)]
    p = jnp.concatenate(pieces, axis=-1)
    return jnp.dot(p.reshape(oh * ow * bt, th * tw * cin), w_ref[...],
                   preferred_element_type=jnp.float32)


def _affine2(acc, aff):
    acc = acc + aff[0:1, :]
    acc = jnp.maximum(acc, 0.0)
    acc = acc * aff[1:2, :] + aff[2:3, :]
    return acc.astype(jnp.bfloat16)


def _net_kernel(xs_ref, w1_ref, a1_ref, w2_ref, a2_ref, w3_ref, a3_ref,
                w4_ref, a4_ref, w5_ref, a5_ref, w6_ref, b6_ref, o_ref):
    bt = xs_ref.shape[2]
    xs = xs_ref[...]

    acc = _conv_cat(xs, 2, 2, 20, 20, w1_ref, 64)
    y = acc.astype(jnp.bfloat16).reshape(20, 20, bt, 32)
    pieces = []
    for r in range(2):
        t = y.reshape(10, 2, 20, bt, 32)[:, r]
        for q in range(2):
            pieces.append(t.reshape(10, 10, 2, bt, 32)[:, :, q])
    y = jnp.concatenate(pieces, axis=-1)
    y = _affine2(y.astype(jnp.float32).reshape(100 * bt, 128), a1_ref[...])
    y = y.reshape(10, 10, bt, 128)
    acc = _conv_cat(y, 2, 2, 9, 9, w2_ref, 128)
    y = _affine2(acc, a2_ref[...]).reshape(9, 9, bt, 64)
    acc = _conv_cat(y, 3, 3, 7, 7, w3_ref, 64)
    y = _affine2(acc, a3_ref[...]).reshape(7, 7, bt, 64)
    y = jnp.pad(y, ((2, 2), (2, 2), (0, 0), (0, 0)))
    acc = _conv_cat(y, 3, 3, 9, 9, w4_ref, 64)
    y = _affine2(acc, a4_ref[...]).reshape(9, 9, bt, 64)
    y = jnp.pad(y, ((1, 1), (1, 1), (0, 0), (0, 0)))
    acc = _conv_cat(y, 2, 2, 10, 10, w5_ref, 64)
    y = _affine2(acc, a5_ref[...]).reshape(10, 10, bt, 128)
    ps = [y[..., i * 32:(i + 1) * 32] for i in range(4)]
    r0 = jnp.stack([ps[0], ps[1]], axis=2).reshape(10, 20, bt, 32)
    r1 = jnp.stack([ps[2], ps[3]], axis=2).reshape(10, 20, bt, 32)
    y = jnp.stack([r0, r1], axis=1).reshape(20, 20, bt, 32)
    y = jnp.pad(y, ((1, 1), (1, 1), (0, 0), (0, 0)))
    z = _conv_cat(y, 2, 2, 21, 21, w6_ref, 32)
    z = z.reshape(21, 21, bt, 16)
    z = jnp.concatenate([z[:, j] for j in range(21)], axis=-1)
    z = z + b6_ref[...][0]
    m = jnp.max(z, axis=(0, 2), keepdims=True)
    e = jnp.exp(z - m)
    s = jnp.sum(e, axis=(0, 2), keepdims=True)
    o_ref[...] = (z - m - jnp.log(s)).astype(o_ref.dtype)


def _pack_affine(bias, g, b, m, v, n_phases=1):
    scale = g / jnp.sqrt(v + _EPS)
    shift = b - m * scale
    rows = jnp.stack([jnp.tile(bias, n_phases), jnp.tile(scale, n_phases),
                      jnp.tile(shift, n_phases)], axis=0)
    return jnp.pad(rows, ((0, 5), (0, 0)))


def _conv_w_s2d(w, s):
    cout, cin, k, _ = w.shape
    wt = jnp.transpose(w, (2, 3, 1, 0))
    wt = wt.reshape(2, s, 2, s, cin, cout)
    wt = jnp.transpose(wt, (0, 2, 1, 3, 4, 5))
    return wt.reshape(4 * s * s * cin, cout)


def _conv_w_flat(w):
    cout, cin, kh, kw = w.shape
    return jnp.transpose(w, (2, 3, 1, 0)).reshape(kh * kw * cin, cout)


def _deconv_w_phase(w, s):
    cin, cout, k, _ = w.shape
    t = k // s
    w6 = w.reshape(cin, cout, t, s, t, s)
    w6 = jnp.flip(w6, axis=(2, 4))
    w6 = jnp.transpose(w6, (2, 4, 0, 3, 5, 1))
    return w6.reshape(t * t * cin, s * s * cout)


def kernel(x, conv1_w, conv1_b, conv2_w, conv2_b, conv3_w, conv3_b,
           deconv1_w, deconv1_b, deconv2_w, deconv2_b, deconv3_w, deconv3_b,
           bn1_g, bn1_b, bn1_m, bn1_v, bn2_g, bn2_b, bn2_m, bn2_v,
           bn3_g, bn3_b, bn3_m, bn3_v, bn4_g, bn4_b, bn4_m, bn4_v,
           bn5_g, bn5_b, bn5_m, bn5_v):
    B = x.shape[0]
    bt = 64 if B % 64 == 0 else (16 if B % 16 == 0 else B)
    bf = jnp.bfloat16

    xs = jnp.transpose(x, (0, 2, 3, 1)).astype(jnp.float32)
    xs = xs.reshape(B, 21, 4, 21, 4, 4)
    xs = jnp.transpose(xs, (1, 3, 0, 2, 4, 5)).reshape(21, 21, B, 64)
    xs = xs.astype(bf)

    w1 = _conv_w_s2d(conv1_w, 4).astype(bf)
    w2 = _conv_w_s2d(conv2_w, 2).astype(bf)
    w3 = _conv_w_flat(conv3_w).astype(bf)
    w4 = _deconv_w_phase(deconv1_w, 1).astype(bf)
    w5 = _deconv_w_phase(deconv2_w, 2).astype(bf)
    w6 = _deconv_w_phase(deconv3_w, 4).astype(bf)
    a1 = _pack_affine(conv1_b, bn1_g, bn1_b, bn1_m, bn1_v, n_phases=4)
    a2 = _pack_affine(conv2_b, bn2_g, bn2_b, bn2_m, bn2_v)
    a3 = _pack_affine(conv3_b, bn3_g, bn3_b, bn3_m, bn3_v)
    a4 = _pack_affine(deconv1_b, bn4_g, bn4_b, bn4_m, bn4_v)
    a5 = _pack_affine(deconv2_b, bn5_g, bn5_b, bn5_m, bn5_v, n_phases=4)
    b6 = jnp.pad(jnp.tile(deconv3_b, 21 * 16)[None, :], ((0, 7), (0, 0)))

    def w_spec(arr):
        return pl.BlockSpec(arr.shape, lambda i: (0,) * arr.ndim)

    z = pl.pallas_call(
        _net_kernel,
        out_shape=jax.ShapeDtypeStruct((21, B, 21 * 16), jnp.float32),
        grid=(B // bt,),
        in_specs=[
            pl.BlockSpec((21, 21, bt, 64), lambda i: (0, 0, i, 0)),
            w_spec(w1), w_spec(a1), w_spec(w2), w_spec(a2),
            w_spec(w3), w_spec(a3), w_spec(w4), w_spec(a4),
            w_spec(w5), w_spec(a5), w_spec(w6), w_spec(b6),
        ],
        out_specs=pl.BlockSpec((21, bt, 21 * 16), lambda i: (0, i, 0)),
        compiler_params=pltpu.CompilerParams(
            dimension_semantics=("parallel",)),
    )(xs, w1, a1, w2, a2, w3, a3, w4, a4, w5, a5, w6, b6)

    z = z.reshape(21, B, 21, 4, 4)
    z = jnp.transpose(z, (1, 0, 3, 2, 4))
    return z.reshape(B, 84, 84)


# speedup vs baseline: 3.3509x; 1.0003x over previous
import jax
import jax.numpy as jnp
from jax.experimental import pallas as pl
from jax.experimental.pallas import tpu as pltpu

_EPS = 1e-5


def _conv_cat(x, th, tw, oh, ow, w_ref, cin):
    bt = x.shape[2]
    pieces = [x[a:a + oh, b:b + ow] for a in range(th) for b in range(tw)]
    p = jnp.concatenate(pieces, axis=-1)
    return jnp.dot(p.reshape(oh * ow * bt, th * tw * cin), w_ref[...],
                   preferred_element_type=jnp.float32)


def _affine2(acc, aff):
    acc = acc + aff[0:1, :]
    acc = jnp.maximum(acc, 0.0)
    acc = acc * aff[1:2, :] + aff[2:3, :]
    return acc.astype(jnp.bfloat16)


def _net_kernel(xs_ref, w1_ref, a1_ref, w2_ref, a2_ref, w3_ref, a3_ref,
                w4_ref, a4_ref, w5_ref, a5_ref, w6_ref, b6_ref, o_ref):
    bt = xs_ref.shape[2]
    xs = xs_ref[...]

    acc = _conv_cat(xs, 2, 2, 20, 20, w1_ref, 64)
    y = acc.astype(jnp.bfloat16).reshape(20, 20, bt, 32)
    pieces = []
    for r in range(2):
        t = y.reshape(10, 2, 20, bt, 32)[:, r]
        for q in range(2):
            pieces.append(t.reshape(10, 10, 2, bt, 32)[:, :, q])
    y = jnp.concatenate(pieces, axis=-1)
    y = _affine2(y.astype(jnp.float32).reshape(100 * bt, 128), a1_ref[...])
    y = y.reshape(10, 10, bt, 128)
    acc = _conv_cat(y, 2, 2, 9, 9, w2_ref, 128)
    y = _affine2(acc, a2_ref[...]).reshape(9, 9, bt, 64)
    acc = _conv_cat(y, 3, 3, 7, 7, w3_ref, 64)
    y = _affine2(acc, a3_ref[...]).reshape(7, 7, bt, 64)
    y = jnp.pad(y, ((2, 2), (2, 2), (0, 0), (0, 0)))
    acc = _conv_cat(y, 3, 3, 9, 9, w4_ref, 64)
    y = _affine2(acc, a4_ref[...]).reshape(9, 9, bt, 64)
    y = jnp.pad(y, ((1, 1), (1, 1), (0, 0), (0, 0)))
    acc = _conv_cat(y, 2, 2, 10, 10, w5_ref, 64)
    y = _affine2(acc, a5_ref[...]).reshape(10, 10, bt, 128)
    ps = [y[..., i * 32:(i + 1) * 32] for i in range(4)]
    r0 = jnp.stack([ps[0], ps[1]], axis=2).reshape(10, 20, bt, 32)
    r1 = jnp.stack([ps[2], ps[3]], axis=2).reshape(10, 20, bt, 32)
    y = jnp.stack([r0, r1], axis=1).reshape(20, 20, bt, 32)
    y = jnp.pad(y, ((1, 1), (1, 1), (0, 0), (0, 0)))
    z = _conv_cat(y, 2, 2, 21, 21, w6_ref, 32)
    z = z.reshape(21, 21, bt, 16)
    z = jnp.concatenate([z[:, j] for j in range(21)], axis=-1)
    z = z + b6_ref[...][0]
    m = jnp.max(z, axis=(0, 2), keepdims=True)
    e = jnp.exp(z - m)
    s = jnp.sum(e, axis=(0, 2), keepdims=True)
    o_ref[...] = (z - m - jnp.log(s)).astype(o_ref.dtype)


def _pack_affine(bias, g, b, m, v, n_phases=1):
    scale = g / jnp.sqrt(v + _EPS)
    shift = b - m * scale
    rows = jnp.stack([jnp.tile(bias, n_phases), jnp.tile(scale, n_phases),
                      jnp.tile(shift, n_phases)], axis=0)
    return jnp.pad(rows, ((0, 5), (0, 0)))


def _conv_w_s2d(w, s):
    cout, cin, k, _ = w.shape
    wt = jnp.transpose(w, (2, 3, 1, 0))
    wt = wt.reshape(2, s, 2, s, cin, cout)
    wt = jnp.transpose(wt, (0, 2, 1, 3, 4, 5))
    return wt.reshape(4 * s * s * cin, cout)


def _conv_w_flat(w):
    cout, cin, kh, kw = w.shape
    return jnp.transpose(w, (2, 3, 1, 0)).reshape(kh * kw * cin, cout)


def _deconv_w_phase(w, s):
    cin, cout, k, _ = w.shape
    t = k // s
    w6 = w.reshape(cin, cout, t, s, t, s)
    w6 = jnp.flip(w6, axis=(2, 4))
    w6 = jnp.transpose(w6, (2, 4, 0, 3, 5, 1))
    return w6.reshape(t * t * cin, s * s * cout)


def kernel(x, conv1_w, conv1_b, conv2_w, conv2_b, conv3_w, conv3_b,
           deconv1_w, deconv1_b, deconv2_w, deconv2_b, deconv3_w, deconv3_b,
           bn1_g, bn1_b, bn1_m, bn1_v, bn2_g, bn2_b, bn2_m, bn2_v,
           bn3_g, bn3_b, bn3_m, bn3_v, bn4_g, bn4_b, bn4_m, bn4_v,
           bn5_g, bn5_b, bn5_m, bn5_v):
    B = x.shape[0]
    bt = 64 if B % 64 == 0 else (16 if B % 16 == 0 else B)
    bf = jnp.bfloat16

    xs = jnp.transpose(x, (0, 2, 3, 1)).astype(jnp.float32)
    xs = xs.reshape(B, 21, 4, 21, 4, 4)
    xs = jnp.transpose(xs, (1, 3, 0, 2, 4, 5)).reshape(21, 21, B, 64)
    xs = xs.astype(bf)

    w1 = _conv_w_s2d(conv1_w, 4).astype(bf)
    w2 = _conv_w_s2d(conv2_w, 2).astype(bf)
    w3 = _conv_w_flat(conv3_w).astype(bf)
    w4 = _deconv_w_phase(deconv1_w, 1).astype(bf)
    w5 = _deconv_w_phase(deconv2_w, 2).astype(bf)
    w6 = _deconv_w_phase(deconv3_w, 4).astype(bf)
    a1 = _pack_affine(conv1_b, bn1_g, bn1_b, bn1_m, bn1_v, n_phases=4)
    a2 = _pack_affine(conv2_b, bn2_g, bn2_b, bn2_m, bn2_v)
    a3 = _pack_affine(conv3_b, bn3_g, bn3_b, bn3_m, bn3_v)
    a4 = _pack_affine(deconv1_b, bn4_g, bn4_b, bn4_m, bn4_v)
    a5 = _pack_affine(deconv2_b, bn5_g, bn5_b, bn5_m, bn5_v, n_phases=4)
    b6 = jnp.pad(jnp.tile(deconv3_b, 21 * 16)[None, :], ((0, 7), (0, 0)))

    def w_spec(arr):
        return pl.BlockSpec(arr.shape, lambda i: (0,) * arr.ndim)

    z = pl.pallas_call(
        _net_kernel,
        out_shape=jax.ShapeDtypeStruct((21, B, 21 * 16), jnp.float32),
        grid=(B // bt,),
        in_specs=[
            pl.BlockSpec((21, 21, bt, 64), lambda i: (0, 0, i, 0)),
            w_spec(w1), w_spec(a1), w_spec(w2), w_spec(a2),
            w_spec(w3), w_spec(a3), w_spec(w4), w_spec(a4),
            w_spec(w5), w_spec(a5), w_spec(w6), w_spec(b6),
        ],
        out_specs=pl.BlockSpec((21, bt, 21 * 16), lambda i: (0, i, 0)),
        compiler_params=pltpu.CompilerParams(
            dimension_semantics=("parallel",)),
    )(xs, w1, a1, w2, a2, w3, a3, w4, a4, w5, a5, w6, b6)

    z = z.reshape(21, B, 21, 4, 4)
    z = jnp.transpose(z, (1, 0, 3, 2, 4))
    return z.reshape(B, 84, 84)
